# baseline (speedup 1.0000x reference)
import jax
import jax.numpy as jnp
from jax.experimental import pallas as pl


def _pos_embed_body(cw_ref, rw_ref, o_ref):
    cwT = cw_ref[...].T
    rwT = rw_ref[...].T
    d, w = cwT.shape
    h = rwT.shape[1]
    top = jnp.broadcast_to(cwT[:, None, :], (d, h, w))
    bot = jnp.broadcast_to(rwT[:, :, None], (d, h, w))
    o_ref[0] = jnp.concatenate([top, bot], axis=0)


def kernel(mask, row_weight, col_weight):
    bs, h, w = mask.shape
    d = row_weight.shape[1]
    out_shape = jax.ShapeDtypeStruct((bs, 2 * d, h, w), jnp.float32)
    return pl.pallas_call(
        _pos_embed_body,
        grid=(bs,),
        in_specs=[
            pl.BlockSpec((w, d), lambda b: (0, 0)),
            pl.BlockSpec((h, d), lambda b: (0, 0)),
        ],
        out_specs=pl.BlockSpec((1, 2 * d, h, w), lambda b: (b, 0, 0, 0)),
        out_shape=out_shape,
    )(col_weight, row_weight)


# speedup vs baseline: 1.0476x; 1.0476x over previous
import jax
import jax.numpy as jnp
from jax.experimental import pallas as pl
from jax.experimental.pallas import tpu as pltpu


def _pos_embed_body(cw_ref, rw_ref, o_ref, slab, sems):
    cwT = cw_ref[...].T
    rwT = rw_ref[...].T
    d, w = cwT.shape
    h = rwT.shape[1]
    slab[0:d] = jnp.broadcast_to(cwT[:, None, :], (d, h, w))
    slab[d:] = jnp.broadcast_to(rwT[:, :, None], (d, h, w))
    bs = o_ref.shape[0]
    for b in range(bs):
        pltpu.make_async_copy(slab, o_ref.at[b], sems.at[b]).start()
    for b in range(bs):
        pltpu.make_async_copy(slab, o_ref.at[b], sems.at[b]).wait()


def kernel(mask, row_weight, col_weight):
    bs, h, w = mask.shape
    d = row_weight.shape[1]
    out_shape = jax.ShapeDtypeStruct((bs, 2 * d, h, w), jnp.float32)
    return pl.pallas_call(
        _pos_embed_body,
        in_specs=[
            pl.BlockSpec(memory_space=pltpu.VMEM),
            pl.BlockSpec(memory_space=pltpu.VMEM),
        ],
        out_specs=pl.BlockSpec(memory_space=pl.ANY),
        out_shape=out_shape,
        scratch_shapes=[
            pltpu.VMEM((2 * d, h, w), jnp.float32),
            pltpu.SemaphoreType.DMA((bs,)),
        ],
    )(col_weight, row_weight)
